# baseline (speedup 1.0000x reference)
import functools

import jax
import jax.numpy as jnp
from jax import lax
from jax.experimental import pallas as pl
from jax.experimental.pallas import tpu as pltpu
from jax.experimental.pallas import tpu_sc as plsc

B = 16384
D = 32

_info = plsc.get_sparse_core_info()
_NC, _NS = _info.num_cores, _info.num_subcores
NW = _NC * _NS
BPW = B // NW
NSLOT = 4
NDMAX = BPW + 32


def _sc_gather(su2d, si2d, pu, pi, ugT, igT, umT, imT):
    mesh = plsc.VectorSubcoreMesh(core_axis_name="c", subcore_axis_name="s")

    @functools.partial(
        pl.kernel,
        mesh=mesh,
        compiler_params=pltpu.CompilerParams(needs_layout_passes=False),
        out_type=[jax.ShapeDtypeStruct((B, 128), jnp.float32)
                  for _ in range(4)],
        scratch_types=[
            pltpu.VMEM((BPW // 128, 128), jnp.int32),
            pltpu.VMEM((BPW // 64, 64), jnp.int32),
            pltpu.VMEM((NDMAX,), jnp.int32),
            pltpu.VMEM((NDMAX,), jnp.int32),
            pltpu.VMEM((NSLOT, 2, D, 128), jnp.float32),
            pltpu.VMEM((4, 64, 128), jnp.float32),
            pltpu.VMEM((4, 64, 128), jnp.float32),
        ] + [pltpu.SemaphoreType.DMA for _ in range(NSLOT)] + [
            pltpu.SemaphoreType.DMA,
        ],
    )
    def k(su_hbm, si_hbm, pu_hbm, pi_hbm, ug_hbm, ig_hbm, um_hbm, im_hbm,
          ug_out, ig_out, um_out, im_out,
          sv_v, pv_v, dcol_v, bnds_v, slabs, stage_g, stage_m,
          sem0, sem1, sem2, sem3, wsem):
        wid = lax.axis_index("s") * _NC + lax.axis_index("c")
        base = pl.multiple_of(wid * BPW, 128)
        sems = (sem0, sem1, sem2, sem3)
        iota = lax.iota(jnp.int32, 16)
        rows_lo = iota
        rows_hi = iota + 16

        for sk_hbm, pk_hbm, tg_hbm, tm_hbm, g_out, m_out in (
                (su_hbm, pu_hbm, ug_hbm, um_hbm, ug_out, um_out),
                (si_hbm, pi_hbm, ig_hbm, im_hbm, ig_out, im_out)):
            pltpu.sync_copy(sk_hbm.at[pl.ds(wid * (BPW // 128), BPW // 128)],
                            sv_v)
            for r in range(BPW // 64):
                pltpu.sync_copy(pk_hbm.at[pl.ds(base + r * 64, 64)],
                                pv_v.at[r])

            carry = jnp.int32(0)
            for c in range(BPW // 16):
                j16 = c * 16 + iota
                cur = sv_v[c // 8, pl.ds((c % 8) * 16, 16)]
                pj = jnp.maximum(j16 - 1, 0)
                prev = plsc.load_gather(sv_v, [pj >> 7, pj & 127])
                flag = ((cur >> 7) != (prev >> 7)) | (j16 == 0)
                s16 = plsc.cumsum(flag.astype(jnp.int32)) + carry
                d16 = s16 - 1
                plsc.store_scatter(dcol_v, [d16], cur & (-128), mask=flag)
                plsc.store_scatter(bnds_v, [d16], j16, mask=flag)
                carry = jnp.max(s16)
            nd = carry
            plsc.store_scatter(bnds_v, [jnp.full((16,), nd, jnp.int32)],
                               jnp.full((16,), BPW, jnp.int32),
                               mask=iota == 0)

            def fire(colbase, slot):
                c0 = pl.multiple_of(colbase, 128)
                for t, tab in enumerate((tg_hbm, tm_hbm)):
                    pltpu.make_async_copy(
                        tab.at[:, pl.ds(c0, 128)],
                        slabs.at[slot, t], sems[slot],
                    ).start()

            def drain(slot):
                for t in range(2):
                    pltpu.make_async_copy(
                        tg_hbm.at[:, pl.ds(0, 128)],
                        slabs.at[slot, t], sems[slot],
                    ).wait()

            dcol0 = plsc.load_gather(dcol_v, [iota])
            for s in range(NSLOT):
                @pl.when(s < nd)
                def _():
                    fire(dcol0[s], s)

            def outer(g, _):
                g16 = g * 16
                dcolv = plsc.load_gather(dcol_v, [g16 + iota])
                dcolv2 = plsc.load_gather(dcol_v, [g16 + 16 + iota])
                b_lo = plsc.load_gather(bnds_v, [g16 + iota])
                b_hi = plsc.load_gather(bnds_v, [g16 + 1 + iota])

                for l in range(16):
                    d = g16 + l
                    slot = l % NSLOT
                    nxt = dcolv[l + NSLOT] if l < 16 - NSLOT \
                        else dcolv2[l - 16 + NSLOT]

                    @pl.when(d < nd)
                    def _():
                        drain(slot)

                        def idxbody(j, _c):
                            lane = plsc.load_gather(
                                sv_v, [jnp.full((16,), j >> 7, jnp.int32),
                                       jnp.full((16,), j & 127, jnp.int32)]
                            ) & 127
                            for rows, o in ((rows_lo, 0), (rows_hi, 16)):
                                gv = plsc.load_gather(
                                    slabs.at[slot, 0], [rows, lane])
                                mv = plsc.load_gather(
                                    slabs.at[slot, 1], [rows, lane])
                                stage_g[(j >> 6) & 3, j & 63,
                                        pl.ds(o, 16)] = gv
                                stage_m[(j >> 6) & 3, j & 63,
                                        pl.ds(o, 16)] = mv

                            @pl.when((j & 63) == 63)
                            def _():
                                f = j >> 6
                                pltpu.async_copy(
                                    stage_g.at[f & 3], g_out.at[pv_v.at[f]],
                                    wsem).start()
                                pltpu.async_copy(
                                    stage_m.at[f & 3], m_out.at[pv_v.at[f]],
                                    wsem).start()

                                @pl.when((f & 3) == 3)
                                def _():
                                    for _i in range(4):
                                        for st in (stage_g, stage_m):
                                            pltpu.make_async_copy(
                                                g_out.at[pl.ds(0, 64)],
                                                st.at[0], wsem).wait()
                            return 0

                        lax.fori_loop(b_lo[l], b_hi[l], idxbody, 0)

                        @pl.when(d + NSLOT < nd)
                        def _():
                            fire(nxt, slot)
                return 0

            lax.fori_loop(0, (nd + 15) >> 4, outer, 0)

    return k(su2d, si2d, pu, pi, ugT, igT, umT, imT)


def _tc_dense(ug128, ig128, um128, im128, w0u, w0i, b0, w1t, b1,
              w2t, b2, w3t, b3, wpg, wph, bp):
    TM = 2048

    def body(ug_r, ig_r, um_r, im_r, w0u_r, w0i_r, b0_r, w1_r, b1_r,
             w2_r, b2_r, w3_r, b3_r, wpg_r, wph_r, bp_r, out_r):
        dot = functools.partial(jnp.dot, preferred_element_type=jnp.float32)
        ug = ug_r[:, :D]
        ig = ig_r[:, :D]
        um = um_r[:, :D]
        im = im_r[:, :D]
        h = dot(um, w0u_r[...]) + dot(im, w0i_r[...]) + b0_r[...]
        h = jnp.maximum(h, 0.0)
        h = jnp.maximum(dot(h, w1_r[...]) + b1_r[...], 0.0)
        h = jnp.maximum(dot(h, w2_r[...]) + b2_r[...], 0.0)
        h = jnp.maximum(dot(h, w3_r[...]) + b3_r[...], 0.0)
        logit = (dot(ug * ig, wpg_r[...]) + dot(h, wph_r[...]) + bp_r[...])
        out_r[...] = 1.0 / (1.0 + jnp.exp(-logit))

    data_spec = pl.BlockSpec((TM, 128), lambda i: (i, 0))
    full = lambda a: pl.BlockSpec(a.shape, lambda i: (0, 0))
    return pl.pallas_call(
        body,
        grid=(B // TM,),
        in_specs=[data_spec, data_spec, data_spec, data_spec,
                  full(w0u), full(w0i), full(b0), full(w1t), full(b1),
                  full(w2t), full(b2), full(w3t), full(b3),
                  full(wpg), full(wph), full(bp)],
        out_specs=pl.BlockSpec((TM, 1), lambda i: (i, 0)),
        out_shape=jax.ShapeDtypeStruct((B, 1), jnp.float32),
    )(ug128, ig128, um128, im128, w0u, w0i, b0, w1t, b1, w2t, b2, w3t, b3,
      wpg, wph, bp)


def kernel(user_indices, item_indices, ue_gmf, ie_gmf, ue_mlp, ie_mlp,
           W0, b0, W1, b1, W2, b2, W3, b3, Wp, bp):
    iota = lax.iota(jnp.int32, B)
    su, pu = lax.sort_key_val(user_indices.astype(jnp.int32), iota)
    si, pi = lax.sort_key_val(item_indices.astype(jnp.int32), iota)
    tTs = [t.T for t in (ue_gmf, ie_gmf, ue_mlp, ie_mlp)]
    ug128, ig128, um128, im128 = _sc_gather(
        su.reshape(B // 128, 128), si.reshape(B // 128, 128), pu, pi, *tTs)
    return _tc_dense(ug128, ig128, um128, im128,
                     W0[:, :D].T, W0[:, D:].T, b0.reshape(1, -1),
                     W1.T, b1.reshape(1, -1), W2.T, b2.reshape(1, -1),
                     W3.T, b3.reshape(1, -1),
                     Wp[:, :D].T, Wp[:, D:].T, bp.reshape(1, 1))


# speedup vs baseline: 1.2054x; 1.2054x over previous
import functools

import jax
import jax.numpy as jnp
from jax import lax
from jax.experimental import pallas as pl
from jax.experimental.pallas import tpu as pltpu
from jax.experimental.pallas import tpu_sc as plsc

B = 16384
D = 32

_info = plsc.get_sparse_core_info()
_NC, _NS = _info.num_cores, _info.num_subcores
NW = _NC * _NS
BPW = B // NW
NSLOT = 8
NDMAX = BPW + 32


def _sc_gather(su2d, si2d, pu, pi, ugT, igT, umT, imT):
    mesh = plsc.VectorSubcoreMesh(core_axis_name="c", subcore_axis_name="s")

    @functools.partial(
        pl.kernel,
        mesh=mesh,
        compiler_params=pltpu.CompilerParams(needs_layout_passes=False),
        out_type=[jax.ShapeDtypeStruct((B, 128), jnp.float32)
                  for _ in range(4)],
        scratch_types=[
            pltpu.VMEM((BPW // 128, 128), jnp.int32),
            pltpu.VMEM((BPW // 64, 64), jnp.int32),
            pltpu.VMEM((NDMAX,), jnp.int32),
            pltpu.VMEM((NDMAX,), jnp.int32),
            pltpu.VMEM((NSLOT, 2, D, 128), jnp.float32),
            pltpu.VMEM((64, 128), jnp.float32),
            pltpu.VMEM((64, 128), jnp.float32),
        ] + [pltpu.SemaphoreType.DMA for _ in range(NSLOT)] + [
            pltpu.SemaphoreType.DMA,
        ],
    )
    def k(su_hbm, si_hbm, pu_hbm, pi_hbm, ug_hbm, ig_hbm, um_hbm, im_hbm,
          ug_out, ig_out, um_out, im_out,
          sv_v, pv_v, dcol_v, bnds_v, slabs, stage_g, stage_m,
          sem0, sem1, sem2, sem3, sem4, sem5, sem6, sem7, wsem):
        wid = lax.axis_index("s") * _NC + lax.axis_index("c")
        base = pl.multiple_of(wid * BPW, 128)
        sems = (sem0, sem1, sem2, sem3, sem4, sem5, sem6, sem7)
        iota = lax.iota(jnp.int32, 16)
        rows_lo = iota
        rows_hi = iota + 16

        for sk_hbm, pk_hbm, tg_hbm, tm_hbm, g_out, m_out in (
                (su_hbm, pu_hbm, ug_hbm, um_hbm, ug_out, um_out),
                (si_hbm, pi_hbm, ig_hbm, im_hbm, ig_out, im_out)):
            pltpu.sync_copy(sk_hbm.at[pl.ds(wid * (BPW // 128), BPW // 128)],
                            sv_v)
            for r in range(BPW // 64):
                pltpu.sync_copy(pk_hbm.at[pl.ds(base + r * 64, 64)],
                                pv_v.at[r])

            carry = jnp.int32(0)
            for c in range(BPW // 16):
                j16 = c * 16 + iota
                cur = sv_v[c // 8, pl.ds((c % 8) * 16, 16)]
                pj = jnp.maximum(j16 - 1, 0)
                prev = plsc.load_gather(sv_v, [pj >> 7, pj & 127])
                flag = ((cur >> 7) != (prev >> 7)) | (j16 == 0)
                s16 = plsc.cumsum(flag.astype(jnp.int32)) + carry
                d16 = s16 - 1
                plsc.store_scatter(dcol_v, [d16], cur & (-128), mask=flag)
                plsc.store_scatter(bnds_v, [d16], j16, mask=flag)
                carry = jnp.max(s16)
            nd = carry
            plsc.store_scatter(bnds_v, [jnp.full((16,), nd, jnp.int32)],
                               jnp.full((16,), BPW, jnp.int32),
                               mask=iota == 0)

            def fire(colbase, slot):
                c0 = pl.multiple_of(colbase, 128)
                for t, tab in enumerate((tg_hbm, tm_hbm)):
                    pltpu.make_async_copy(
                        tab.at[:, pl.ds(c0, 128)],
                        slabs.at[slot, t], sems[slot],
                    ).start()

            def drain(slot):
                for t in range(2):
                    pltpu.make_async_copy(
                        tg_hbm.at[:, pl.ds(0, 128)],
                        slabs.at[slot, t], sems[slot],
                    ).wait()

            dcol0 = plsc.load_gather(dcol_v, [iota])
            for s in range(NSLOT):
                @pl.when(s < nd)
                def _():
                    fire(dcol0[s], s)

            def outer(g, _):
                g16 = g * 16
                dcolv = plsc.load_gather(dcol_v, [g16 + iota])
                dcolv2 = plsc.load_gather(dcol_v, [g16 + 16 + iota])
                b_lo = plsc.load_gather(bnds_v, [g16 + iota])
                b_hi = plsc.load_gather(bnds_v, [g16 + 1 + iota])

                for l in range(16):
                    d = g16 + l
                    slot = l % NSLOT
                    nxt = dcolv[l + NSLOT] if l < 16 - NSLOT \
                        else dcolv2[l - 16 + NSLOT]

                    @pl.when(d < nd)
                    def _():
                        drain(slot)

                        def idxbody(j, _c):
                            lane = plsc.load_gather(
                                sv_v, [jnp.full((16,), j >> 7, jnp.int32),
                                       jnp.full((16,), j & 127, jnp.int32)]
                            ) & 127
                            for rows, o in ((rows_lo, 0), (rows_hi, 16)):
                                gv = plsc.load_gather(
                                    slabs.at[slot, 0], [rows, lane])
                                mv = plsc.load_gather(
                                    slabs.at[slot, 1], [rows, lane])
                                stage_g[j & 63, pl.ds(o, 16)] = gv
                                stage_m[j & 63, pl.ds(o, 16)] = mv

                            @pl.when((j & 63) == 63)
                            def _():
                                f = j >> 6
                                w1 = pltpu.async_copy(
                                    stage_g, g_out.at[pv_v.at[f]], wsem)
                                w2 = pltpu.async_copy(
                                    stage_m, m_out.at[pv_v.at[f]], wsem)
                                w1.wait()
                                w2.wait()
                            return 0

                        lax.fori_loop(b_lo[l], b_hi[l], idxbody, 0)

                        @pl.when(d + NSLOT < nd)
                        def _():
                            fire(nxt, slot)
                return 0

            lax.fori_loop(0, (nd + 15) >> 4, outer, 0)

    return k(su2d, si2d, pu, pi, ugT, igT, umT, imT)


def _tc_dense(ug128, ig128, um128, im128, w0u, w0i, b0, w1t, b1,
              w2t, b2, w3t, b3, wpg, wph, bp):
    TM = 2048

    def body(ug_r, ig_r, um_r, im_r, w0u_r, w0i_r, b0_r, w1_r, b1_r,
             w2_r, b2_r, w3_r, b3_r, wpg_r, wph_r, bp_r, out_r):
        dot = functools.partial(jnp.dot, preferred_element_type=jnp.float32)
        ug = ug_r[:, :D]
        ig = ig_r[:, :D]
        um = um_r[:, :D]
        im = im_r[:, :D]
        h = dot(um, w0u_r[...]) + dot(im, w0i_r[...]) + b0_r[...]
        h = jnp.maximum(h, 0.0)
        h = jnp.maximum(dot(h, w1_r[...]) + b1_r[...], 0.0)
        h = jnp.maximum(dot(h, w2_r[...]) + b2_r[...], 0.0)
        h = jnp.maximum(dot(h, w3_r[...]) + b3_r[...], 0.0)
        logit = (dot(ug * ig, wpg_r[...]) + dot(h, wph_r[...]) + bp_r[...])
        out_r[...] = 1.0 / (1.0 + jnp.exp(-logit))

    data_spec = pl.BlockSpec((TM, 128), lambda i: (i, 0))
    full = lambda a: pl.BlockSpec(a.shape, lambda i: (0, 0))
    return pl.pallas_call(
        body,
        grid=(B // TM,),
        in_specs=[data_spec, data_spec, data_spec, data_spec,
                  full(w0u), full(w0i), full(b0), full(w1t), full(b1),
                  full(w2t), full(b2), full(w3t), full(b3),
                  full(wpg), full(wph), full(bp)],
        out_specs=pl.BlockSpec((TM, 1), lambda i: (i, 0)),
        out_shape=jax.ShapeDtypeStruct((B, 1), jnp.float32),
    )(ug128, ig128, um128, im128, w0u, w0i, b0, w1t, b1, w2t, b2, w3t, b3,
      wpg, wph, bp)


def kernel(user_indices, item_indices, ue_gmf, ie_gmf, ue_mlp, ie_mlp,
           W0, b0, W1, b1, W2, b2, W3, b3, Wp, bp):
    iota = lax.iota(jnp.int32, B)
    su, pu = lax.sort_key_val(user_indices.astype(jnp.int32), iota)
    si, pi = lax.sort_key_val(item_indices.astype(jnp.int32), iota)
    tTs = [t.T for t in (ue_gmf, ie_gmf, ue_mlp, ie_mlp)]
    ug128, ig128, um128, im128 = _sc_gather(
        su.reshape(B // 128, 128), si.reshape(B // 128, 128), pu, pi, *tTs)
    return _tc_dense(ug128, ig128, um128, im128,
                     W0[:, :D].T, W0[:, D:].T, b0.reshape(1, -1),
                     W1.T, b1.reshape(1, -1), W2.T, b2.reshape(1, -1),
                     W3.T, b3.reshape(1, -1),
                     Wp[:, :D].T, Wp[:, D:].T, bp.reshape(1, 1))


# speedup vs baseline: 1.2287x; 1.0193x over previous
import functools

import jax
import jax.numpy as jnp
from jax import lax
from jax.experimental import pallas as pl
from jax.experimental.pallas import tpu as pltpu
from jax.experimental.pallas import tpu_sc as plsc

B = 16384
D = 32

_info = plsc.get_sparse_core_info()
_NC, _NS = _info.num_cores, _info.num_subcores
NW = _NC * _NS
BPW = B // NW
NSLOT = 8
NDMAX = BPW + 32


def _sc_gather(su2d, si2d, pu, pi, ugT, igT, umT, imT):
    mesh = plsc.VectorSubcoreMesh(core_axis_name="c", subcore_axis_name="s")

    @functools.partial(
        pl.kernel,
        mesh=mesh,
        compiler_params=pltpu.CompilerParams(needs_layout_passes=False),
        out_type=[jax.ShapeDtypeStruct((B, 128), jnp.float32)
                  for _ in range(4)],
        scratch_types=[
            pltpu.VMEM((BPW // 128, 128), jnp.int32),
            pltpu.VMEM((BPW // 128, 128), jnp.int32),
            pltpu.VMEM((NDMAX,), jnp.int32),
            pltpu.VMEM((NDMAX,), jnp.int32),
            pltpu.VMEM((NSLOT, 2, D, 128), jnp.float32),
            pltpu.VMEM((128, 128), jnp.float32),
            pltpu.VMEM((128, 128), jnp.float32),
        ] + [pltpu.SemaphoreType.DMA for _ in range(NSLOT)] + [
            pltpu.SemaphoreType.DMA,
        ],
    )
    def k(su_hbm, si_hbm, pu_hbm, pi_hbm, ug_hbm, ig_hbm, um_hbm, im_hbm,
          ug_out, ig_out, um_out, im_out,
          sv_v, pv_v, dcol_v, bnds_v, slabs, stage_g, stage_m,
          sem0, sem1, sem2, sem3, sem4, sem5, sem6, sem7, wsem):
        wid = lax.axis_index("s") * _NC + lax.axis_index("c")
        base = pl.multiple_of(wid * BPW, 128)
        sems = (sem0, sem1, sem2, sem3, sem4, sem5, sem6, sem7)
        iota = lax.iota(jnp.int32, 16)
        rows_lo = iota
        rows_hi = iota + 16

        for sk_hbm, pk_hbm, tg_hbm, tm_hbm, g_out, m_out in (
                (su_hbm, pu_hbm, ug_hbm, um_hbm, ug_out, um_out),
                (si_hbm, pi_hbm, ig_hbm, im_hbm, ig_out, im_out)):
            pltpu.sync_copy(sk_hbm.at[pl.ds(wid * (BPW // 128), BPW // 128)],
                            sv_v)
            for r in range(BPW // 128):
                pltpu.sync_copy(pk_hbm.at[pl.ds(base + r * 128, 128)],
                                pv_v.at[r])

            carry = jnp.int32(0)
            for c in range(BPW // 16):
                j16 = c * 16 + iota
                cur = sv_v[c // 8, pl.ds((c % 8) * 16, 16)]
                pj = jnp.maximum(j16 - 1, 0)
                prev = plsc.load_gather(sv_v, [pj >> 7, pj & 127])
                flag = ((cur >> 7) != (prev >> 7)) | (j16 == 0)
                s16 = plsc.cumsum(flag.astype(jnp.int32)) + carry
                d16 = s16 - 1
                plsc.store_scatter(dcol_v, [d16], cur & (-128), mask=flag)
                plsc.store_scatter(bnds_v, [d16], j16, mask=flag)
                carry = jnp.max(s16)
            nd = carry
            plsc.store_scatter(bnds_v, [jnp.full((16,), nd, jnp.int32)],
                               jnp.full((16,), BPW, jnp.int32),
                               mask=iota == 0)

            def fire(colbase, slot):
                c0 = pl.multiple_of(colbase, 128)
                for t, tab in enumerate((tg_hbm, tm_hbm)):
                    pltpu.make_async_copy(
                        tab.at[:, pl.ds(c0, 128)],
                        slabs.at[slot, t], sems[slot],
                    ).start()

            def drain(slot):
                for t in range(2):
                    pltpu.make_async_copy(
                        tg_hbm.at[:, pl.ds(0, 128)],
                        slabs.at[slot, t], sems[slot],
                    ).wait()

            dcol0 = plsc.load_gather(dcol_v, [iota])
            for s in range(NSLOT):
                @pl.when(s < nd)
                def _():
                    fire(dcol0[s], s)

            def outer(g, _):
                g16 = g * 16
                dcolv = plsc.load_gather(dcol_v, [g16 + iota])
                dcolv2 = plsc.load_gather(dcol_v, [g16 + 16 + iota])
                b_lo = plsc.load_gather(bnds_v, [g16 + iota])
                b_hi = plsc.load_gather(bnds_v, [g16 + 1 + iota])

                for l in range(16):
                    d = g16 + l
                    slot = l % NSLOT
                    nxt = dcolv[l + NSLOT] if l < 16 - NSLOT \
                        else dcolv2[l - 16 + NSLOT]

                    @pl.when(d < nd)
                    def _():
                        drain(slot)

                        def idxbody(j, _c):
                            lane = plsc.load_gather(
                                sv_v, [jnp.full((16,), j >> 7, jnp.int32),
                                       jnp.full((16,), j & 127, jnp.int32)]
                            ) & 127
                            for rows, o in ((rows_lo, 0), (rows_hi, 16)):
                                gv = plsc.load_gather(
                                    slabs.at[slot, 0], [rows, lane])
                                mv = plsc.load_gather(
                                    slabs.at[slot, 1], [rows, lane])
                                stage_g[j & 127, pl.ds(o, 16)] = gv
                                stage_m[j & 127, pl.ds(o, 16)] = mv

                            @pl.when((j & 127) == 127)
                            def _():
                                f = j >> 7
                                w1 = pltpu.async_copy(
                                    stage_g, g_out.at[pv_v.at[f]], wsem)
                                w2 = pltpu.async_copy(
                                    stage_m, m_out.at[pv_v.at[f]], wsem)
                                w1.wait()
                                w2.wait()
                            return 0

                        lax.fori_loop(b_lo[l], b_hi[l], idxbody, 0)

                        @pl.when(d + NSLOT < nd)
                        def _():
                            fire(nxt, slot)
                return 0

            lax.fori_loop(0, (nd + 15) >> 4, outer, 0)

    return k(su2d, si2d, pu, pi, ugT, igT, umT, imT)


def _tc_dense(ug128, ig128, um128, im128, w0u, w0i, b0, w1t, b1,
              w2t, b2, w3t, b3, wpg, wph, bp):
    TM = 2048

    def body(ug_r, ig_r, um_r, im_r, w0u_r, w0i_r, b0_r, w1_r, b1_r,
             w2_r, b2_r, w3_r, b3_r, wpg_r, wph_r, bp_r, out_r):
        dot = functools.partial(jnp.dot, preferred_element_type=jnp.float32)
        ug = ug_r[:, :D]
        ig = ig_r[:, :D]
        um = um_r[:, :D]
        im = im_r[:, :D]
        h = dot(um, w0u_r[...]) + dot(im, w0i_r[...]) + b0_r[...]
        h = jnp.maximum(h, 0.0)
        h = jnp.maximum(dot(h, w1_r[...]) + b1_r[...], 0.0)
        h = jnp.maximum(dot(h, w2_r[...]) + b2_r[...], 0.0)
        h = jnp.maximum(dot(h, w3_r[...]) + b3_r[...], 0.0)
        logit = (dot(ug * ig, wpg_r[...]) + dot(h, wph_r[...]) + bp_r[...])
        out_r[...] = 1.0 / (1.0 + jnp.exp(-logit))

    data_spec = pl.BlockSpec((TM, 128), lambda i: (i, 0))
    full = lambda a: pl.BlockSpec(a.shape, lambda i: (0, 0))
    return pl.pallas_call(
        body,
        grid=(B // TM,),
        in_specs=[data_spec, data_spec, data_spec, data_spec,
                  full(w0u), full(w0i), full(b0), full(w1t), full(b1),
                  full(w2t), full(b2), full(w3t), full(b3),
                  full(wpg), full(wph), full(bp)],
        out_specs=pl.BlockSpec((TM, 1), lambda i: (i, 0)),
        out_shape=jax.ShapeDtypeStruct((B, 1), jnp.float32),
    )(ug128, ig128, um128, im128, w0u, w0i, b0, w1t, b1, w2t, b2, w3t, b3,
      wpg, wph, bp)


def kernel(user_indices, item_indices, ue_gmf, ie_gmf, ue_mlp, ie_mlp,
           W0, b0, W1, b1, W2, b2, W3, b3, Wp, bp):
    iota = lax.iota(jnp.int32, B)
    su, pu = lax.sort_key_val(user_indices.astype(jnp.int32), iota)
    si, pi = lax.sort_key_val(item_indices.astype(jnp.int32), iota)
    tTs = [t.T for t in (ue_gmf, ie_gmf, ue_mlp, ie_mlp)]
    ug128, ig128, um128, im128 = _sc_gather(
        su.reshape(B // 128, 128), si.reshape(B // 128, 128), pu, pi, *tTs)
    return _tc_dense(ug128, ig128, um128, im128,
                     W0[:, :D].T, W0[:, D:].T, b0.reshape(1, -1),
                     W1.T, b1.reshape(1, -1), W2.T, b2.reshape(1, -1),
                     W3.T, b3.reshape(1, -1),
                     Wp[:, :D].T, Wp[:, D:].T, bp.reshape(1, 1))
